# baseline (speedup 1.0000x reference)
import functools

import jax
import jax.numpy as jnp
from jax import lax
from jax.experimental import pallas as pl
from jax.experimental.pallas import tpu as pltpu
from jax.experimental.pallas import tpu_sc as plsc

N = 16384
NW = 32
C = 4096
NB = 4
STEPS = 30
LANES = 16


def _sc_matvec_kernel(nch):
    mesh = plsc.VectorSubcoreMesh(core_axis_name="c", subcore_axis_name="s")

    @functools.partial(
        pl.kernel,
        mesh=mesh,
        out_type=jax.ShapeDtypeStruct((2, N), jnp.float32),
        compiler_params=pltpu.CompilerParams(needs_layout_passes=False),
        scratch_types=[
            pltpu.VMEM_SHARED((16, N), jnp.float32),
            pltpu.VMEM((16, N // 16), jnp.float32),
            pltpu.VMEM((N,), jnp.float32),
            pltpu.VMEM((N,), jnp.float32),
            pltpu.VMEM((C,), jnp.int32),
            pltpu.VMEM((C,), jnp.int32),
            pltpu.VMEM((C,), jnp.int32),
            pltpu.VMEM((C,), jnp.int32),
            pltpu.VMEM((C,), jnp.float32),
            pltpu.VMEM((C,), jnp.float32),
            pltpu.VMEM((C,), jnp.float32),
            pltpu.VMEM((C,), jnp.float32),
            pltpu.SemaphoreType.DMA,
            pltpu.SemaphoreType.DMA,
            pltpu.SemaphoreType.DMA,
            pltpu.SemaphoreType.DMA,
            pltpu.SemaphoreType.DMA,
        ],
    )
    def matvec(idx_hbm, val_hbm, p_hbm, out_hbm, grid_s, red_v, p_v, y_v,
               ibuf0, ibuf1, ibuf2, ibuf3, vbuf0, vbuf1, vbuf2, vbuf3,
               sem0, sem1, sem2, sem3, psem):
        ibufs = (ibuf0, ibuf1, ibuf2, ibuf3)
        vbufs = (vbuf0, vbuf1, vbuf2, vbuf3)
        cid = lax.axis_index("c")
        sid = lax.axis_index("s")
        wid = sid * 2 + cid
        base_chunk = wid * nch

        sems = (sem0, sem1, sem2, sem3)

        def start(ch, b):
            pltpu.make_async_copy(
                idx_hbm.at[pl.ds((base_chunk + ch) * C, C)], ibufs[b], sems[b]
            ).start()
            pltpu.make_async_copy(
                val_hbm.at[pl.ds((base_chunk + ch) * C, C)], vbufs[b], sems[b]
            ).start()

        def wait(b):
            pltpu.make_async_copy(
                idx_hbm.at[pl.ds(0, C)], ibufs[b], sems[b]
            ).wait()
            pltpu.make_async_copy(
                val_hbm.at[pl.ds(0, C)], vbufs[b], sems[b]
            ).wait()

        def process(iref, vref):
            G = 8

            def inner(j, carry):
                base = j * (G * LANES)
                idx = [iref[pl.ds(base + k * LANES, LANES)] for k in range(G)]
                val = [vref[pl.ds(base + k * LANES, LANES)] for k in range(G)]
                rows = [v & 0x3FFF for v in idx]
                cols = [lax.shift_right_logical(v, 14) for v in idx]
                pv1 = [plsc.load_gather(p_v, [c]) for c in cols]
                pv2 = [plsc.load_gather(p_v, [r]) for r in rows]
                for k in range(G):
                    plsc.addupdate_scatter(y_v, [rows[k]], val[k] * pv1[k])
                for k in range(G):
                    plsc.addupdate_scatter(y_v, [cols[k]], val[k] * pv2[k])
                return carry

            lax.fori_loop(0, C // (G * LANES), inner, 0)

        for b in range(NB - 1):
            start(b, b)
        pltpu.make_async_copy(p_hbm, p_v, psem).start()

        def zbody(i, carry):
            y_v[pl.ds(i * LANES, LANES)] = jnp.zeros((LANES,), jnp.float32)
            return carry

        lax.fori_loop(0, N // LANES, zbody, 0, unroll=8)
        pltpu.make_async_copy(p_hbm, p_v, psem).wait()

        def outer(i, carry):
            g = i * NB
            for b in range(NB):
                ch = g + b

                @pl.when(ch + NB - 1 < nch)
                def _():
                    start(ch + NB - 1, (b + NB - 1) % NB)

                wait(b)
                process(ibufs[b], vbufs[b])
            return carry

        lax.fori_loop(0, nch // NB, outer, 0)

        sl = N // 16
        pltpu.sync_copy(y_v, grid_s.at[sid])
        plsc.subcore_barrier()
        pltpu.sync_copy(grid_s.at[:, pl.ds(sid * sl, sl)], red_v)

        def rbody(j, carry):
            acc = red_v[0, pl.ds(j * LANES, LANES)]
            for t in range(1, 16):
                acc = acc + red_v[t, pl.ds(j * LANES, LANES)]
            y_v[pl.ds(j * LANES, LANES)] = acc
            return carry

        lax.fori_loop(0, sl // LANES, rbody, 0)
        pltpu.sync_copy(
            y_v.at[pl.ds(0, sl)], out_hbm.at[cid, pl.ds(sid * sl, sl)]
        )

    return matvec


def _tc_update(partial, p, r, x):
    def body(partial_ref, p_ref, r_ref, x_ref, xo_ref, ro_ref, po_ref):
        ap = jnp.sum(partial_ref[...], axis=0) + 200.0 * p_ref[...]
        pv = p_ref[...]
        rv = r_ref[...]
        xv = x_ref[...]
        rs = jnp.sum(rv * rv)
        pap = jnp.sum(pv * ap)
        alpha = rs / (pap + 1e-12)
        xn = xv + alpha * pv
        rn = rv - alpha * ap
        rs_new = jnp.sum(rn * rn)
        beta = rs_new / (rs + 1e-12)
        pn = rn + beta * pv
        xo_ref[...] = xn
        ro_ref[...] = rn
        po_ref[...] = pn

    shp = jax.ShapeDtypeStruct((128, 128), jnp.float32)
    return pl.pallas_call(
        body,
        out_shape=(shp, shp, shp),
    )(partial, p, r, x)


def kernel(A_rows, A_cols, A_values, b):
    n = b.shape[0]
    i32 = jnp.int32
    rows = A_rows.astype(i32)
    cols = A_cols.astype(i32)
    vals = 0.5 * A_values

    e = rows.shape[0]
    per_worker = NW * C
    nch = -(-e // per_worker)
    nch = -(-nch // NB) * NB
    e_pad = NW * nch * C
    pad = e_pad - e
    if pad:
        pad_idx = jnp.arange(pad, dtype=i32) % jnp.int32(n)
        rows = jnp.concatenate([rows, pad_idx])
        cols = jnp.concatenate([cols, pad_idx])
        vals = jnp.concatenate([vals, jnp.zeros((pad,), jnp.float32)])

    packed_idx = jnp.left_shift(cols, 14) | rows
    packed_val = vals

    matvec = _sc_matvec_kernel(nch)

    b2 = b.reshape(128, 128)
    x0 = jnp.zeros((128, 128), jnp.float32)

    def step(_, carry):
        x, r, p = carry
        part = matvec(packed_idx, packed_val, p.reshape(-1))
        x, r, p = _tc_update(part.reshape(2, 128, 128), p, r, x)
        return (x, r, p)

    x, _, _ = lax.fori_loop(0, STEPS, step, (x0, b2, b2))
    return x.reshape(-1)


# speedup vs baseline: 1.0347x; 1.0347x over previous
import functools

import jax
import jax.numpy as jnp
from jax import lax
from jax.experimental import pallas as pl
from jax.experimental.pallas import tpu as pltpu
from jax.experimental.pallas import tpu_sc as plsc

N = 16384
NW = 32
C = 4096
NB = 4
STEPS = 30
LANES = 16


def _sc_matvec_kernel(nch):
    mesh = plsc.VectorSubcoreMesh(core_axis_name="c", subcore_axis_name="s")

    @functools.partial(
        pl.kernel,
        mesh=mesh,
        out_type=jax.ShapeDtypeStruct((2, N), jnp.float32),
        compiler_params=pltpu.CompilerParams(needs_layout_passes=False),
        scratch_types=[
            pltpu.VMEM_SHARED((16, N), jnp.float32),
            pltpu.VMEM((16, N // 16), jnp.float32),
            pltpu.VMEM((N,), jnp.float32),
            pltpu.VMEM((N,), jnp.float32),
            pltpu.VMEM((C,), jnp.int32),
            pltpu.VMEM((C,), jnp.int32),
            pltpu.VMEM((C,), jnp.int32),
            pltpu.VMEM((C,), jnp.int32),
            pltpu.VMEM((C,), jnp.float32),
            pltpu.VMEM((C,), jnp.float32),
            pltpu.VMEM((C,), jnp.float32),
            pltpu.VMEM((C,), jnp.float32),
            pltpu.SemaphoreType.DMA,
            pltpu.SemaphoreType.DMA,
            pltpu.SemaphoreType.DMA,
            pltpu.SemaphoreType.DMA,
        ],
    )
    def matvec(idx_hbm, val_hbm, p_hbm, out_hbm, grid_s, red_v, p_v, y_v,
               ibuf0, ibuf1, ibuf2, ibuf3, vbuf0, vbuf1, vbuf2, vbuf3,
               sem0, sem1, sem2, sem3):
        ibufs = (ibuf0, ibuf1, ibuf2, ibuf3)
        vbufs = (vbuf0, vbuf1, vbuf2, vbuf3)
        cid = lax.axis_index("c")
        sid = lax.axis_index("s")
        wid = sid * 2 + cid
        base_chunk = wid * nch

        sems = (sem0, sem1, sem2, sem3)

        def start(ch, b):
            pltpu.make_async_copy(
                idx_hbm.at[pl.ds((base_chunk + ch) * C, C)], ibufs[b], sems[b]
            ).start()
            pltpu.make_async_copy(
                val_hbm.at[pl.ds((base_chunk + ch) * C, C)], vbufs[b], sems[b]
            ).start()

        def wait(b):
            pltpu.make_async_copy(
                idx_hbm.at[pl.ds(0, C)], ibufs[b], sems[b]
            ).wait()
            pltpu.make_async_copy(
                val_hbm.at[pl.ds(0, C)], vbufs[b], sems[b]
            ).wait()

        def process(iref, vref):
            G = 8

            def inner(j, carry):
                base = j * (G * LANES)
                idx = [iref[pl.ds(base + k * LANES, LANES)] for k in range(G)]
                val = [vref[pl.ds(base + k * LANES, LANES)] for k in range(G)]
                rows = [v & 0x3FFF for v in idx]
                cols = [lax.shift_right_logical(v, 14) for v in idx]
                pv1 = [plsc.load_gather(p_v, [c]) for c in cols]
                pv2 = [plsc.load_gather(p_v, [r]) for r in rows]
                for k in range(G):
                    plsc.addupdate_scatter(y_v, [rows[k]], val[k] * pv1[k])
                for k in range(G):
                    plsc.addupdate_scatter(y_v, [cols[k]], val[k] * pv2[k])
                return carry

            lax.fori_loop(0, C // (G * LANES), inner, 0)

        for b in range(NB - 1):
            start(b, b)

        @pl.when(sid == 0)
        def _():
            pltpu.sync_copy(p_hbm, grid_s.at[0])

        def zbody(i, carry):
            y_v[pl.ds(i * LANES, LANES)] = jnp.zeros((LANES,), jnp.float32)
            return carry

        lax.fori_loop(0, N // LANES, zbody, 0, unroll=8)
        plsc.subcore_barrier()
        pltpu.sync_copy(grid_s.at[0], p_v)
        plsc.subcore_barrier()

        def outer(i, carry):
            g = i * NB
            for b in range(NB):
                ch = g + b

                @pl.when(ch + NB - 1 < nch)
                def _():
                    start(ch + NB - 1, (b + NB - 1) % NB)

                wait(b)
                process(ibufs[b], vbufs[b])
            return carry

        lax.fori_loop(0, nch // NB, outer, 0)

        sl = N // 16
        pltpu.sync_copy(y_v, grid_s.at[sid])
        plsc.subcore_barrier()
        pltpu.sync_copy(grid_s.at[:, pl.ds(sid * sl, sl)], red_v)

        def rbody(j, carry):
            acc = red_v[0, pl.ds(j * LANES, LANES)]
            for t in range(1, 16):
                acc = acc + red_v[t, pl.ds(j * LANES, LANES)]
            y_v[pl.ds(j * LANES, LANES)] = acc
            return carry

        lax.fori_loop(0, sl // LANES, rbody, 0)
        pltpu.sync_copy(
            y_v.at[pl.ds(0, sl)], out_hbm.at[cid, pl.ds(sid * sl, sl)]
        )

    return matvec


def _tc_update(partial, p, r, x):
    def body(partial_ref, p_ref, r_ref, x_ref, xo_ref, ro_ref, po_ref):
        ap = jnp.sum(partial_ref[...], axis=0) + 200.0 * p_ref[...]
        pv = p_ref[...]
        rv = r_ref[...]
        xv = x_ref[...]
        rs = jnp.sum(rv * rv)
        pap = jnp.sum(pv * ap)
        alpha = rs / (pap + 1e-12)
        xn = xv + alpha * pv
        rn = rv - alpha * ap
        rs_new = jnp.sum(rn * rn)
        beta = rs_new / (rs + 1e-12)
        pn = rn + beta * pv
        xo_ref[...] = xn
        ro_ref[...] = rn
        po_ref[...] = pn

    shp = jax.ShapeDtypeStruct((128, 128), jnp.float32)
    return pl.pallas_call(
        body,
        out_shape=(shp, shp, shp),
    )(partial, p, r, x)


def kernel(A_rows, A_cols, A_values, b):
    n = b.shape[0]
    i32 = jnp.int32
    rows = A_rows.astype(i32)
    cols = A_cols.astype(i32)
    vals = 0.5 * A_values

    e = rows.shape[0]
    per_worker = NW * C
    nch = -(-e // per_worker)
    nch = -(-nch // NB) * NB
    e_pad = NW * nch * C
    pad = e_pad - e
    if pad:
        pad_idx = jnp.arange(pad, dtype=i32) % jnp.int32(n)
        rows = jnp.concatenate([rows, pad_idx])
        cols = jnp.concatenate([cols, pad_idx])
        vals = jnp.concatenate([vals, jnp.zeros((pad,), jnp.float32)])

    packed_idx = jnp.left_shift(cols, 14) | rows
    packed_val = vals

    matvec = _sc_matvec_kernel(nch)

    b2 = b.reshape(128, 128)
    x0 = jnp.zeros((128, 128), jnp.float32)

    def step(_, carry):
        x, r, p = carry
        part = matvec(packed_idx, packed_val, p.reshape(-1))
        x, r, p = _tc_update(part.reshape(2, 128, 128), p, r, x)
        return (x, r, p)

    x, _, _ = lax.fori_loop(0, STEPS, step, (x0, b2, b2))
    return x.reshape(-1)
